# Initial kernel scaffold; baseline (speedup 1.0000x reference)
import jax
import jax.numpy as jnp
from jax.experimental import pallas as pl


def kernel(x, edge_index, adj_values, W):
    raise NotImplementedError("write your pallas kernel here")


# speedup vs baseline: 2.9028x; 2.9028x over previous
import functools

import jax
import jax.numpy as jnp
from jax import lax
from jax.experimental import pallas as pl
from jax.experimental.pallas import tpu as pltpu
from jax.experimental.pallas import tpu_sc as plsc

N = 10000
E = 160000
D = 256
DH = 128

NUM_TILES = 16
K = 80
EDGES_PER_TILE = E // NUM_TILES
CHUNKS = EDGES_PER_TILE // K
NP = 10240
ROWS_PER_TILE = NP // NUM_TILES
RB = 128
ROW_BLOCKS = ROWS_PER_TILE // RB

MM_ROWS = 2000


def _mm_body(x_ref, w_ref, o_ref):
    o_ref[...] = jnp.dot(x_ref[...], w_ref[...],
                         preferred_element_type=jnp.float32)


def _matmul_split(x, W):
    n_rb = N // MM_ROWS
    return pl.pallas_call(
        _mm_body,
        grid=(2, n_rb),
        in_specs=[
            pl.BlockSpec((MM_ROWS, D), lambda c, r: (r, 0)),
            pl.BlockSpec((D, DH), lambda c, r: (0, c)),
        ],
        out_specs=pl.BlockSpec((MM_ROWS, DH), lambda c, r, _n=n_rb: (c * _n + r, 0)),
        out_shape=jax.ShapeDtypeStruct((2 * N, DH), jnp.float32),
    )(x, W)


_mesh = plsc.VectorSubcoreMesh(core_axis_name="c", subcore_axis_name="s")


@functools.partial(
    pl.kernel,
    out_type=jax.ShapeDtypeStruct((2, NP, DH), jnp.float32),
    mesh=_mesh,
    scratch_types=[
        pltpu.VMEM((K,), jnp.int32),
        pltpu.VMEM((K,), jnp.int32),
        pltpu.VMEM((K,), jnp.float32),
        pltpu.VMEM((K,), jnp.int32),
        pltpu.VMEM((K, DH), jnp.float32),
        pltpu.VMEM((RB, DH), jnp.float32),
        pltpu.VMEM_SHARED((NP, DH), jnp.float32),
        pltpu.SemaphoreType.DMA,
    ],
)
def _sc_aggregate(ht_hbm, src_hbm, dst_hbm, adj_hbm, out_hbm,
                  src_v, dst_v, adj_v, idx_v, rows_v, stg_v, agg_sh, sem):
    c = lax.axis_index("c")
    s = lax.axis_index("s")

    def _zero_row(r, carry):
        for j in range(DH // 16):
            stg_v[r, pl.ds(j * 16, 16)] = jnp.zeros((16,), jnp.float32)
        return carry
    lax.fori_loop(0, RB, _zero_row, 0)
    for b in range(ROW_BLOCKS):
        pltpu.sync_copy(stg_v, agg_sh.at[pl.ds(s * ROWS_PER_TILE + b * RB, RB)])
    plsc.subcore_barrier()

    base0 = s * EDGES_PER_TILE
    row_off = c * N

    def _chunk(k, carry):
        base = base0 + k * K
        pltpu.sync_copy(src_hbm.at[pl.ds(base, K)], src_v)
        pltpu.sync_copy(dst_hbm.at[pl.ds(base, K)], dst_v)
        pltpu.sync_copy(adj_hbm.at[pl.ds(base, K)], adj_v)
        for j in range(K // 16):
            idx_v[pl.ds(j * 16, 16)] = src_v[pl.ds(j * 16, 16)] + row_off
        pltpu.async_copy(ht_hbm.at[idx_v], rows_v, sem).wait()

        def _scale(g, inner):
            a16 = adj_v[pl.ds(g * 16, 16)]
            for lane in range(16):
                e = g * 16 + lane
                a = a16[lane]
                for j in range(DH // 16):
                    rows_v[e, pl.ds(j * 16, 16)] = rows_v[e, pl.ds(j * 16, 16)] * a
            return inner
        lax.fori_loop(0, K // 16, _scale, 0)

        pltpu.sync_copy(rows_v, agg_sh.at[dst_v], add=True)
        return carry
    lax.fori_loop(0, CHUNKS, _chunk, 0)
    plsc.subcore_barrier()

    for b in range(ROW_BLOCKS):
        r0 = s * ROWS_PER_TILE + b * RB
        pltpu.sync_copy(agg_sh.at[pl.ds(r0, RB)], stg_v)

        def _relu_row(r, carry):
            for j in range(DH // 16):
                v = stg_v[r, pl.ds(j * 16, 16)]
                stg_v[r, pl.ds(j * 16, 16)] = jnp.maximum(v, 0.0)
            return carry
        lax.fori_loop(0, RB, _relu_row, 0)
        pltpu.sync_copy(stg_v, out_hbm.at[c, pl.ds(r0, RB)])


def kernel(x, edge_index, adj_values, W):
    ht = _matmul_split(x, W)
    src = edge_index[0]
    dst = edge_index[1]
    agg = _sc_aggregate(ht, src, dst, adj_values)
    return jnp.concatenate([agg[0, :N], agg[1, :N]], axis=1)
